# Initial kernel scaffold; baseline (speedup 1.0000x reference)
import jax
import jax.numpy as jnp
from jax.experimental import pallas as pl


def kernel(x, edge_attr, edge_index, params):
    raise NotImplementedError("write your pallas kernel here")


# speedup vs baseline: 1.2118x; 1.2118x over previous
import functools

import jax
import jax.numpy as jnp
from jax import lax
from jax.experimental import pallas as pl
from jax.experimental.pallas import tpu as pltpu
from jax.experimental.pallas import tpu_sc as plsc

N = 10000
E = 160000
DIN = 128
DE = 16
HID = 64
OD = 16
OD2 = OD * OD

NC, NS, LN = 2, 16, 16
NW = NC * NS
CH = 128
EP = 163840
PW = EP // NW
NCH = PW // CH
NP = 10112
ZR = NP // NS

BQ = 2048

_HI = jax.lax.Precision.HIGHEST


def _mesh():
    return plsc.VectorSubcoreMesh(core_axis_name="c", subcore_axis_name="s")


_SC_PARAMS = pltpu.CompilerParams(use_tc_tiling_on_sc=False)


def _sc_gather(table, idx2d):
    @functools.partial(
        pl.kernel,
        out_type=jax.ShapeDtypeStruct((EP, LN), jnp.float32),
        mesh=_mesh(),
        compiler_params=_SC_PARAMS,
        scratch_types=[
            pltpu.VMEM((NCH, CH), jnp.int32),
            pltpu.VMEM((PW, LN), jnp.float32),
            pltpu.SemaphoreType.DMA,
            pltpu.SemaphoreType.DMA,
        ],
    )
    def k(table_hbm, idx_hbm, out_hbm, idx_v, rows_v, isem, gsem):
        wid = lax.axis_index("s") * NC + lax.axis_index("c")
        pltpu.make_async_copy(
            idx_hbm.at[pl.ds(wid * NCH, NCH)], idx_v, isem).start()
        pltpu.make_async_copy(
            idx_hbm.at[pl.ds(wid * NCH, NCH)], idx_v, isem).wait()

        @pl.loop(0, NCH)
        def _fire(j):
            pltpu.make_async_copy(
                table_hbm.at[idx_v.at[j]],
                rows_v.at[pl.ds(j * CH, CH)], gsem).start()

        @pl.loop(0, NCH)
        def _drain(j):
            pltpu.make_async_copy(
                table_hbm.at[idx_v.at[j]],
                rows_v.at[pl.ds(j * CH, CH)], gsem).wait()

        pltpu.make_async_copy(
            rows_v, out_hbm.at[pl.ds(wid * PW, PW)], isem).start()
        pltpu.make_async_copy(
            rows_v, out_hbm.at[pl.ds(wid * PW, PW)], isem).wait()

    return k(table, idx2d)


def _sc_scatter_add(rows, idx2d):
    @functools.partial(
        pl.kernel,
        out_type=jax.ShapeDtypeStruct((NC, NP, LN), jnp.float32),
        mesh=_mesh(),
        compiler_params=_SC_PARAMS,
        scratch_types=[
            pltpu.VMEM((NCH, CH), jnp.int32),
            pltpu.VMEM((PW, LN), jnp.float32),
            pltpu.VMEM((ZR, LN), jnp.float32),
            pltpu.VMEM_SHARED((NP, LN), jnp.float32),
            pltpu.SemaphoreType.DMA,
        ],
    )
    def k(rows_hbm, idx_hbm, out_hbm, idx_v, rows_v, zb_v, acc_sh, sem):
        c = lax.axis_index("c")
        s = lax.axis_index("s")
        wid = s * NC + c

        @pl.loop(0, ZR)
        def _zero(r):
            zb_v[r, :] = jnp.zeros((LN,), jnp.float32)

        pltpu.sync_copy(zb_v, acc_sh.at[pl.ds(s * ZR, ZR)])
        plsc.subcore_barrier()

        pltpu.sync_copy(idx_hbm.at[pl.ds(wid * NCH, NCH)], idx_v)
        pltpu.sync_copy(rows_hbm.at[pl.ds(wid * PW, PW)], rows_v)

        @pl.loop(0, NCH)
        def _scat(j):
            pltpu.sync_copy(
                rows_v.at[pl.ds(j * CH, CH)],
                acc_sh.at[idx_v.at[j]], add=True)

        plsc.subcore_barrier()
        pltpu.sync_copy(
            acc_sh.at[pl.ds(s * ZR, ZR)],
            out_hbm.at[c, pl.ds(s * ZR, ZR)])

    return k(rows, idx2d)


def _sc_degree(idx2d):
    @functools.partial(
        pl.kernel,
        out_type=jax.ShapeDtypeStruct((NC, NP, LN), jnp.float32),
        mesh=_mesh(),
        compiler_params=_SC_PARAMS,
        scratch_types=[
            pltpu.VMEM((NCH, CH), jnp.int32),
            pltpu.VMEM((CH, LN), jnp.float32),
            pltpu.VMEM((ZR, LN), jnp.float32),
            pltpu.VMEM_SHARED((NP, LN), jnp.float32),
            pltpu.SemaphoreType.DMA,
        ],
    )
    def k(idx_hbm, out_hbm, idx_v, ones_v, zb_v, acc_sh, sem):
        c = lax.axis_index("c")
        s = lax.axis_index("s")
        wid = s * NC + c

        @pl.loop(0, ZR)
        def _zero(r):
            zb_v[r, :] = jnp.zeros((LN,), jnp.float32)

        @pl.loop(0, CH)
        def _ones(r):
            ones_v[r, :] = jnp.ones((LN,), jnp.float32)

        pltpu.sync_copy(zb_v, acc_sh.at[pl.ds(s * ZR, ZR)])
        plsc.subcore_barrier()

        pltpu.sync_copy(idx_hbm.at[pl.ds(wid * NCH, NCH)], idx_v)

        @pl.loop(0, NCH)
        def _scat(j):
            pltpu.sync_copy(ones_v, acc_sh.at[idx_v.at[j]], add=True)

        plsc.subcore_barrier()
        pltpu.sync_copy(
            acc_sh.at[pl.ds(s * ZR, ZR)],
            out_hbm.at[c, pl.ds(s * ZR, ZR)])

    return k(idx2d)


def _tc_proj0(x, Wp, bp, degp):
    def body(x_ref, wp_ref, bp_ref, degp_ref, xp_ref, dinv_ref):
        xp_ref[...] = jnp.dot(
            x_ref[...], wp_ref[...],
            preferred_element_type=jnp.float32, precision=_HI) + bp_ref[...]
        d = degp_ref[0, :N, :] + degp_ref[1, :N, :]
        dinv_ref[...] = 1.0 / jnp.maximum(d, 1.0)

    return pl.pallas_call(
        body,
        out_shape=(jax.ShapeDtypeStruct((N, OD), jnp.float32),
                   jax.ShapeDtypeStruct((N, OD), jnp.float32)),
    )(x, Wp, bp.reshape(1, OD), degp)


def _tc_msg(ea_p, xg, We1, be1, We2, be2, Kmat, Smat):
    def body(ea_ref, xg_ref, w1_ref, b1_ref, w2_ref, b2_ref,
             k_ref, s_ref, out_ref):
        wh = jnp.maximum(
            jnp.dot(ea_ref[...], w1_ref[...],
                    preferred_element_type=jnp.float32, precision=_HI)
            + b1_ref[...], 0.0)
        we = jnp.dot(wh, w2_ref[...],
                     preferred_element_type=jnp.float32, precision=_HI) \
            + b2_ref[...]
        xr = jnp.dot(xg_ref[...], k_ref[...],
                     preferred_element_type=jnp.float32, precision=_HI)
        out_ref[...] = jnp.dot(xr * we, s_ref[...],
                               preferred_element_type=jnp.float32,
                               precision=_HI)

    return pl.pallas_call(
        body,
        grid=(EP // BQ,),
        in_specs=[
            pl.BlockSpec((BQ, DE), lambda i: (i, 0)),
            pl.BlockSpec((BQ, OD), lambda i: (i, 0)),
            pl.BlockSpec((DE, HID), lambda i: (0, 0)),
            pl.BlockSpec((1, HID), lambda i: (0, 0)),
            pl.BlockSpec((HID, OD2), lambda i: (0, 0)),
            pl.BlockSpec((1, OD2), lambda i: (0, 0)),
            pl.BlockSpec((OD, OD2), lambda i: (0, 0)),
            pl.BlockSpec((OD2, OD), lambda i: (0, 0)),
        ],
        out_specs=pl.BlockSpec((BQ, OD), lambda i: (i, 0)),
        out_shape=jax.ShapeDtypeStruct((EP, OD), jnp.float32),
    )(ea_p, xg, We1, be1.reshape(1, HID), We2, be2.reshape(1, OD2),
      Kmat, Smat)


def _tc_combine(pp, dinv, xp, root, bias, Wpn=None, bpn=None):
    def body(pp_ref, dinv_ref, xp_ref, root_ref, bias_ref, *rest):
        agg = (pp_ref[0, :N, :] + pp_ref[1, :N, :]) * dinv_ref[...]
        conv = agg + jnp.dot(
            xp_ref[...], root_ref[...],
            preferred_element_type=jnp.float32, precision=_HI) + bias_ref[...]
        h = jnp.maximum(conv, 0.0) + xp_ref[...]
        if Wpn is None:
            (out_ref,) = rest
            out_ref[...] = h
        else:
            wn_ref, bn_ref, out_ref = rest
            out_ref[...] = jnp.dot(
                h, wn_ref[...],
                preferred_element_type=jnp.float32,
                precision=_HI) + bn_ref[...]

    args = [pp, dinv, xp, root, bias.reshape(1, OD)]
    if Wpn is not None:
        args += [Wpn, bpn.reshape(1, OD)]
    return pl.pallas_call(
        body,
        out_shape=jax.ShapeDtypeStruct((N, OD), jnp.float32),
    )(*args)


def _tc_edge_mlp(hr, hc, W1, b1, W2, b2):
    def body(hr_ref, hc_ref, w1_ref, b1_ref, w2_ref, b2_ref, out_ref):
        er = hr_ref[...] + hc_ref[...]
        hid = jnp.maximum(
            jnp.dot(er, w1_ref[...],
                    preferred_element_type=jnp.float32, precision=_HI)
            + b1_ref[...], 0.0)
        out_ref[...] = jnp.dot(
            hid, w2_ref[...],
            preferred_element_type=jnp.float32, precision=_HI) + b2_ref[...]

    return pl.pallas_call(
        body,
        grid=(EP // BQ,),
        in_specs=[
            pl.BlockSpec((BQ, OD), lambda i: (i, 0)),
            pl.BlockSpec((BQ, OD), lambda i: (i, 0)),
            pl.BlockSpec((OD, OD), lambda i: (0, 0)),
            pl.BlockSpec((1, OD), lambda i: (0, 0)),
            pl.BlockSpec((OD, 1), lambda i: (0, 0)),
            pl.BlockSpec((1, 1), lambda i: (0, 0)),
        ],
        out_specs=pl.BlockSpec((BQ, 1), lambda i: (i, 0)),
        out_shape=jax.ShapeDtypeStruct((EP, 1), jnp.float32),
    )(hr, hc, W1, b1.reshape(1, OD), W2, b2.reshape(1, 1))


def kernel(x, edge_attr, edge_index, params):
    src = edge_index[0]
    dst = edge_index[1]
    pads = EP - E
    zpad = jnp.zeros((pads,), jnp.int32)
    src_p = jnp.concatenate([src, zpad]).reshape(EP // CH, CH)
    dst_scat = jnp.concatenate(
        [dst, jnp.full((pads,), N, jnp.int32)]).reshape(EP // CH, CH)
    col_p = jnp.concatenate([dst, zpad]).reshape(EP // CH, CH)
    ea_p = jnp.pad(edge_attr, ((0, pads), (0, 0)))

    Kmat = (jnp.arange(OD2)[None, :] // OD
            == jnp.arange(OD)[:, None]).astype(jnp.float32)
    Smat = (jnp.arange(OD2)[:, None] % OD
            == jnp.arange(OD)[None, :]).astype(jnp.float32)

    degp = _sc_degree(dst_scat)
    layers = params["layers"]
    xp, dinv = _tc_proj0(x, layers[0]["Wp"], layers[0]["bp"], degp)

    h = None
    for li, lp in enumerate(layers):
        xg = _sc_gather(xp, src_p)
        msg = _tc_msg(ea_p, xg, lp["We1"], lp["be1"], lp["We2"], lp["be2"],
                      Kmat, Smat)
        pp = _sc_scatter_add(msg, dst_scat)
        if li + 1 < len(layers):
            nxt = layers[li + 1]
            xp = _tc_combine(pp, dinv, xp, lp["root"], lp["bias"],
                             nxt["Wp"], nxt["bp"])
        else:
            h = _tc_combine(pp, dinv, xp, lp["root"], lp["bias"])

    hr = _sc_gather(h, src_p)
    hc = _sc_gather(h, col_p)
    mp = params["edge_mlp"]
    out = _tc_edge_mlp(hr, hc, mp["W1"], mp["b1"], mp["W2"], mp["b2"])
    return out[:E]


# speedup vs baseline: 2.9033x; 2.3960x over previous
import functools

import jax
import jax.numpy as jnp
from jax import lax
from jax.experimental import pallas as pl
from jax.experimental.pallas import tpu as pltpu
from jax.experimental.pallas import tpu_sc as plsc

N = 10000
E = 160000
DIN = 128
DE = 16
HID = 64
OD = 16
OD2 = OD * OD

NC, NS, LN = 2, 16, 16
NW = NC * NS
CH = 128
EP = 163840
PW = EP // NW
NCH = PW // CH
NP = 10112
ZR = NP // NS

BQ = 2048

_HI = jax.lax.Precision.DEFAULT


def _mesh():
    return plsc.VectorSubcoreMesh(core_axis_name="c", subcore_axis_name="s")


_SC_PARAMS = pltpu.CompilerParams(use_tc_tiling_on_sc=False)


def _sc_gather(table, idx2d):
    @functools.partial(
        pl.kernel,
        out_type=jax.ShapeDtypeStruct((EP, LN), jnp.float32),
        mesh=_mesh(),
        compiler_params=_SC_PARAMS,
        scratch_types=[
            pltpu.VMEM((NCH, CH), jnp.int32),
            pltpu.VMEM((PW, LN), jnp.float32),
            pltpu.SemaphoreType.DMA,
            pltpu.SemaphoreType.DMA,
        ],
    )
    def k(table_hbm, idx_hbm, out_hbm, idx_v, rows_v, isem, gsem):
        wid = lax.axis_index("s") * NC + lax.axis_index("c")
        pltpu.make_async_copy(
            idx_hbm.at[pl.ds(wid * NCH, NCH)], idx_v, isem).start()
        pltpu.make_async_copy(
            idx_hbm.at[pl.ds(wid * NCH, NCH)], idx_v, isem).wait()

        @pl.loop(0, NCH)
        def _fire(j):
            pltpu.make_async_copy(
                table_hbm.at[idx_v.at[j]],
                rows_v.at[pl.ds(j * CH, CH)], gsem).start()

        @pl.loop(0, NCH)
        def _drain(j):
            pltpu.make_async_copy(
                table_hbm.at[idx_v.at[j]],
                rows_v.at[pl.ds(j * CH, CH)], gsem).wait()

        pltpu.make_async_copy(
            rows_v, out_hbm.at[pl.ds(wid * PW, PW)], isem).start()
        pltpu.make_async_copy(
            rows_v, out_hbm.at[pl.ds(wid * PW, PW)], isem).wait()

    return k(table, idx2d)


def _sc_scatter_add(rows, idx2d):
    @functools.partial(
        pl.kernel,
        out_type=jax.ShapeDtypeStruct((NC, NP, LN), jnp.float32),
        mesh=_mesh(),
        compiler_params=_SC_PARAMS,
        scratch_types=[
            pltpu.VMEM((NCH, CH), jnp.int32),
            pltpu.VMEM((PW, LN), jnp.float32),
            pltpu.VMEM((ZR, LN), jnp.float32),
            pltpu.VMEM_SHARED((NP, LN), jnp.float32),
            pltpu.SemaphoreType.DMA,
        ],
    )
    def k(rows_hbm, idx_hbm, out_hbm, idx_v, rows_v, zb_v, acc_sh, sem):
        c = lax.axis_index("c")
        s = lax.axis_index("s")
        wid = s * NC + c

        @pl.loop(0, ZR)
        def _zero(r):
            zb_v[r, :] = jnp.zeros((LN,), jnp.float32)

        pltpu.sync_copy(zb_v, acc_sh.at[pl.ds(s * ZR, ZR)])
        plsc.subcore_barrier()

        pltpu.sync_copy(idx_hbm.at[pl.ds(wid * NCH, NCH)], idx_v)
        pltpu.sync_copy(rows_hbm.at[pl.ds(wid * PW, PW)], rows_v)

        @pl.loop(0, NCH)
        def _scat(j):
            pltpu.sync_copy(
                rows_v.at[pl.ds(j * CH, CH)],
                acc_sh.at[idx_v.at[j]], add=True)

        plsc.subcore_barrier()
        pltpu.sync_copy(
            acc_sh.at[pl.ds(s * ZR, ZR)],
            out_hbm.at[c, pl.ds(s * ZR, ZR)])

    return k(rows, idx2d)


def _sc_degree(idx2d):
    @functools.partial(
        pl.kernel,
        out_type=jax.ShapeDtypeStruct((NC, NP, LN), jnp.float32),
        mesh=_mesh(),
        compiler_params=_SC_PARAMS,
        scratch_types=[
            pltpu.VMEM((NCH, CH), jnp.int32),
            pltpu.VMEM((CH, LN), jnp.float32),
            pltpu.VMEM((ZR, LN), jnp.float32),
            pltpu.VMEM_SHARED((NP, LN), jnp.float32),
            pltpu.SemaphoreType.DMA,
        ],
    )
    def k(idx_hbm, out_hbm, idx_v, ones_v, zb_v, acc_sh, sem):
        c = lax.axis_index("c")
        s = lax.axis_index("s")
        wid = s * NC + c

        @pl.loop(0, ZR)
        def _zero(r):
            zb_v[r, :] = jnp.zeros((LN,), jnp.float32)

        @pl.loop(0, CH)
        def _ones(r):
            ones_v[r, :] = jnp.ones((LN,), jnp.float32)

        pltpu.sync_copy(zb_v, acc_sh.at[pl.ds(s * ZR, ZR)])
        plsc.subcore_barrier()

        pltpu.sync_copy(idx_hbm.at[pl.ds(wid * NCH, NCH)], idx_v)

        @pl.loop(0, NCH)
        def _scat(j):
            pltpu.sync_copy(ones_v, acc_sh.at[idx_v.at[j]], add=True)

        plsc.subcore_barrier()
        pltpu.sync_copy(
            acc_sh.at[pl.ds(s * ZR, ZR)],
            out_hbm.at[c, pl.ds(s * ZR, ZR)])

    return k(idx2d)


def _tc_proj0(x, Wp, bp, degp):
    def body(x_ref, wp_ref, bp_ref, degp_ref, xp_ref, dinv_ref):
        xp_ref[...] = jnp.dot(
            x_ref[...], wp_ref[...],
            preferred_element_type=jnp.float32, precision=_HI) + bp_ref[...]
        d = degp_ref[0, :N, :] + degp_ref[1, :N, :]
        dinv_ref[...] = 1.0 / jnp.maximum(d, 1.0)

    return pl.pallas_call(
        body,
        out_shape=(jax.ShapeDtypeStruct((N, OD), jnp.float32),
                   jax.ShapeDtypeStruct((N, OD), jnp.float32)),
    )(x, Wp, bp.reshape(1, OD), degp)


def _tc_msg(ea_p, xg, We1, be1, We2, be2, Kmat, Smat):
    def body(ea_ref, xg_ref, w1_ref, b1_ref, w2_ref, b2_ref,
             k_ref, s_ref, out_ref):
        wh = jnp.maximum(
            jnp.dot(ea_ref[...], w1_ref[...],
                    preferred_element_type=jnp.float32, precision=_HI)
            + b1_ref[...], 0.0)
        we = jnp.dot(wh, w2_ref[...],
                     preferred_element_type=jnp.float32, precision=_HI) \
            + b2_ref[...]
        xr = jnp.dot(xg_ref[...], k_ref[...],
                     preferred_element_type=jnp.float32, precision=_HI)
        out_ref[...] = jnp.dot(xr * we, s_ref[...],
                               preferred_element_type=jnp.float32,
                               precision=_HI)

    return pl.pallas_call(
        body,
        grid=(EP // BQ,),
        in_specs=[
            pl.BlockSpec((BQ, DE), lambda i: (i, 0)),
            pl.BlockSpec((BQ, OD), lambda i: (i, 0)),
            pl.BlockSpec((DE, HID), lambda i: (0, 0)),
            pl.BlockSpec((1, HID), lambda i: (0, 0)),
            pl.BlockSpec((HID, OD2), lambda i: (0, 0)),
            pl.BlockSpec((1, OD2), lambda i: (0, 0)),
            pl.BlockSpec((OD, OD2), lambda i: (0, 0)),
            pl.BlockSpec((OD2, OD), lambda i: (0, 0)),
        ],
        out_specs=pl.BlockSpec((BQ, OD), lambda i: (i, 0)),
        out_shape=jax.ShapeDtypeStruct((EP, OD), jnp.float32),
    )(ea_p, xg, We1, be1.reshape(1, HID), We2, be2.reshape(1, OD2),
      Kmat, Smat)


def _tc_combine(pp, dinv, xp, root, bias, Wpn=None, bpn=None):
    def body(pp_ref, dinv_ref, xp_ref, root_ref, bias_ref, *rest):
        agg = (pp_ref[0, :N, :] + pp_ref[1, :N, :]) * dinv_ref[...]
        conv = agg + jnp.dot(
            xp_ref[...], root_ref[...],
            preferred_element_type=jnp.float32, precision=_HI) + bias_ref[...]
        h = jnp.maximum(conv, 0.0) + xp_ref[...]
        if Wpn is None:
            (out_ref,) = rest
            out_ref[...] = h
        else:
            wn_ref, bn_ref, out_ref = rest
            out_ref[...] = jnp.dot(
                h, wn_ref[...],
                preferred_element_type=jnp.float32,
                precision=_HI) + bn_ref[...]

    args = [pp, dinv, xp, root, bias.reshape(1, OD)]
    if Wpn is not None:
        args += [Wpn, bpn.reshape(1, OD)]
    return pl.pallas_call(
        body,
        out_shape=jax.ShapeDtypeStruct((N, OD), jnp.float32),
    )(*args)


def _tc_edge_mlp(hr, hc, W1, b1, W2, b2):
    def body(hr_ref, hc_ref, w1_ref, b1_ref, w2_ref, b2_ref, out_ref):
        er = hr_ref[...] + hc_ref[...]
        hid = jnp.maximum(
            jnp.dot(er, w1_ref[...],
                    preferred_element_type=jnp.float32, precision=_HI)
            + b1_ref[...], 0.0)
        out_ref[...] = jnp.dot(
            hid, w2_ref[...],
            preferred_element_type=jnp.float32, precision=_HI) + b2_ref[...]

    return pl.pallas_call(
        body,
        grid=(EP // BQ,),
        in_specs=[
            pl.BlockSpec((BQ, OD), lambda i: (i, 0)),
            pl.BlockSpec((BQ, OD), lambda i: (i, 0)),
            pl.BlockSpec((OD, OD), lambda i: (0, 0)),
            pl.BlockSpec((1, OD), lambda i: (0, 0)),
            pl.BlockSpec((OD, 1), lambda i: (0, 0)),
            pl.BlockSpec((1, 1), lambda i: (0, 0)),
        ],
        out_specs=pl.BlockSpec((BQ, 1), lambda i: (i, 0)),
        out_shape=jax.ShapeDtypeStruct((EP, 1), jnp.float32),
    )(hr, hc, W1, b1.reshape(1, OD), W2, b2.reshape(1, 1))


def kernel(x, edge_attr, edge_index, params):
    src = edge_index[0]
    dst = edge_index[1]
    pads = EP - E
    zpad = jnp.zeros((pads,), jnp.int32)
    src_p = jnp.concatenate([src, zpad]).reshape(EP // CH, CH)
    dst_scat = jnp.concatenate(
        [dst, jnp.full((pads,), N, jnp.int32)]).reshape(EP // CH, CH)
    col_p = jnp.concatenate([dst, zpad]).reshape(EP // CH, CH)
    ea_p = jnp.pad(edge_attr, ((0, pads), (0, 0)))

    Kmat = (jnp.arange(OD2)[None, :] // OD
            == jnp.arange(OD)[:, None]).astype(jnp.float32)
    Smat = (jnp.arange(OD2)[:, None] % OD
            == jnp.arange(OD)[None, :]).astype(jnp.float32)

    degp = _sc_degree(dst_scat)
    layers = params["layers"]
    xp, dinv = _tc_proj0(x, layers[0]["Wp"], layers[0]["bp"], degp)

    h = None
    for li, lp in enumerate(layers):
        xg = _sc_gather(xp, src_p)
        msg = _tc_msg(ea_p, xg, lp["We1"], lp["be1"], lp["We2"], lp["be2"],
                      Kmat, Smat)
        pp = _sc_scatter_add(msg, dst_scat)
        if li + 1 < len(layers):
            nxt = layers[li + 1]
            xp = _tc_combine(pp, dinv, xp, lp["root"], lp["bias"],
                             nxt["Wp"], nxt["bp"])
        else:
            h = _tc_combine(pp, dinv, xp, lp["root"], lp["bias"])

    hr = _sc_gather(h, src_p)
    hc = _sc_gather(h, col_p)
    mp = params["edge_mlp"]
    out = _tc_edge_mlp(hr, hc, mp["W1"], mp["b1"], mp["W2"], mp["b2"])
    return out[:E]
